# Initial kernel scaffold; baseline (speedup 1.0000x reference)
import jax
import jax.numpy as jnp
from jax.experimental import pallas as pl


def kernel(x, edge_index, W1, b1, W2, b2):
    raise NotImplementedError("write your pallas kernel here")


# speedup vs baseline: 6.5580x; 6.5580x over previous
import functools

import jax
import jax.numpy as jnp
from jax import lax
from jax.experimental import pallas as pl
from jax.experimental.pallas import tpu as pltpu, tpu_sc as plsc

_N = 10000
_E = 320000
_D = 128

_MESH = plsc.VectorSubcoreMesh(core_axis_name="c", subcore_axis_name="s")
_NC = _MESH.num_cores
_NS = _MESH.num_subcores
_NW = _NC * _NS
_CH = 80
_EPW = _E // _NW
_NCHUNK = _EPW // _CH
_NPAD = 10240
_ZROWS = _NPAD // _NS


@functools.partial(
    pl.kernel,
    out_type=jax.ShapeDtypeStruct((_NC, _NPAD, 16), jnp.float32),
    mesh=_MESH,
    compiler_params=pltpu.CompilerParams(use_tc_tiling_on_sc=False),
    scratch_types=[
        pltpu.VMEM((_NCHUNK, _CH), jnp.int32),
        pltpu.VMEM((_NCHUNK, _CH), jnp.int32),
        pltpu.VMEM((_CH, 16), jnp.float32),
        pltpu.VMEM((_CH, 16), jnp.float32),
        pltpu.VMEM_SHARED((_NPAD, 16), jnp.float32),
    ],
)
def _deg_kernel(srcr, dstr, zdeg, out, src_v, dst_v, po, pi, table):
    cid = lax.axis_index("c")
    sid = lax.axis_index("s")
    wid = sid * _NC + cid
    pltpu.sync_copy(srcr.at[wid], src_v)
    pltpu.sync_copy(dstr.at[wid], dst_v)
    pltpu.sync_copy(zdeg, table.at[pl.ds(sid * _ZROWS, _ZROWS)])

    lane = lax.iota(jnp.int32, 16)
    po_vec = jnp.where(lane < 8, 1.0, 0.0).astype(jnp.float32)
    pi_vec = jnp.where(lane >= 8, 1.0, 0.0).astype(jnp.float32)

    def fill(i, _):
        po[i, pl.ds(0, 16)] = po_vec
        pi[i, pl.ds(0, 16)] = pi_vec
        return 0

    lax.fori_loop(0, _CH, fill, 0)
    plsc.subcore_barrier()

    def chunk(j, _):
        pltpu.sync_copy(po, table.at[src_v.at[j]], add=True)
        pltpu.sync_copy(pi, table.at[dst_v.at[j]], add=True)
        return 0

    lax.fori_loop(0, _NCHUNK, chunk, 0)
    plsc.subcore_barrier()
    pltpu.sync_copy(
        table.at[pl.ds(sid * _ZROWS, _ZROWS)],
        out.at[cid, pl.ds(sid * _ZROWS, _ZROWS)],
    )


@functools.partial(
    pl.kernel,
    out_type=jax.ShapeDtypeStruct((_NC, _NPAD, _D), jnp.float32),
    mesh=_MESH,
    compiler_params=pltpu.CompilerParams(use_tc_tiling_on_sc=False),
    scratch_types=[
        pltpu.VMEM((_NCHUNK, _CH), jnp.int32),
        pltpu.VMEM((_NCHUNK, _CH), jnp.int32),
        pltpu.VMEM((_CH, _D), jnp.float32),
        pltpu.VMEM_SHARED((_NPAD, _D), jnp.float32),
    ],
)
def _agg_kernel(h, srcr, dstr, zrows, out, src_v, dst_v, gbuf, acc):
    cid = lax.axis_index("c")
    sid = lax.axis_index("s")
    wid = sid * _NC + cid
    pltpu.sync_copy(srcr.at[wid], src_v)
    pltpu.sync_copy(dstr.at[wid], dst_v)
    pltpu.sync_copy(zrows, acc.at[pl.ds(sid * _ZROWS, _ZROWS)])
    plsc.subcore_barrier()

    def chunk(j, _):
        pltpu.sync_copy(h.at[src_v.at[j]], gbuf)
        pltpu.sync_copy(gbuf, acc.at[dst_v.at[j]], add=True)
        return 0

    lax.fori_loop(0, _NCHUNK, chunk, 0)
    plsc.subcore_barrier()
    pltpu.sync_copy(
        acc.at[pl.ds(sid * _ZROWS, _ZROWS)],
        out.at[cid, pl.ds(sid * _ZROWS, _ZROWS)],
    )


def _norm_body(degs_ref, x_ref, h_ref, nin_ref, nout_ref):
    d_out = degs_ref[0, :, 0:1] + degs_ref[1, :, 0:1]
    d_in = degs_ref[0, :, 8:9] + degs_ref[1, :, 8:9]
    n_out = jnp.where(d_out > 0, lax.rsqrt(jnp.maximum(d_out, 1.0)), 0.0)
    n_in = jnp.where(d_in > 0, lax.rsqrt(jnp.maximum(d_in, 1.0)), 0.0)
    h_ref[...] = x_ref[...] * n_out
    nin_ref[...] = n_in
    nout_ref[...] = n_out


def _layer_body(p_ref, nin_ref, nout_ref, w_ref, b_ref, o_ref, *, relu):
    a = (p_ref[0] + p_ref[1]) * nin_ref[...]
    o = jnp.dot(a, w_ref[...], preferred_element_type=jnp.float32) + b_ref[...]
    if relu:
        o = jnp.maximum(o, 0.0) * nout_ref[...]
    o_ref[...] = o


_BN = 1000


def _norm_call(degs, x):
    return pl.pallas_call(
        _norm_body,
        grid=(_N // _BN,),
        in_specs=[
            pl.BlockSpec((_NC, _BN, 16), lambda i: (0, i, 0)),
            pl.BlockSpec((_BN, _D), lambda i: (i, 0)),
        ],
        out_specs=[
            pl.BlockSpec((_BN, _D), lambda i: (i, 0)),
            pl.BlockSpec((_BN, 1), lambda i: (i, 0)),
            pl.BlockSpec((_BN, 1), lambda i: (i, 0)),
        ],
        out_shape=[
            jax.ShapeDtypeStruct((_N, _D), jnp.float32),
            jax.ShapeDtypeStruct((_N, 1), jnp.float32),
            jax.ShapeDtypeStruct((_N, 1), jnp.float32),
        ],
    )(degs, x)


def _layer_call(p, n_in, n_out, wm, bm, relu):
    return pl.pallas_call(
        functools.partial(_layer_body, relu=relu),
        grid=(_N // _BN,),
        in_specs=[
            pl.BlockSpec((_NC, _BN, _D), lambda i: (0, i, 0)),
            pl.BlockSpec((_BN, 1), lambda i: (i, 0)),
            pl.BlockSpec((_BN, 1), lambda i: (i, 0)),
            pl.BlockSpec((_D, _D), lambda i: (0, 0)),
            pl.BlockSpec((1, _D), lambda i: (0, 0)),
        ],
        out_specs=pl.BlockSpec((_BN, _D), lambda i: (i, 0)),
        out_shape=jax.ShapeDtypeStruct((_N, _D), jnp.float32),
    )(p, n_in, n_out, wm, bm)


def kernel(x, edge_index, W1, b1, W2, b2):
    src_r = edge_index[0].reshape(_NW, _NCHUNK, _CH)
    dst_r = edge_index[1].reshape(_NW, _NCHUNK, _CH)
    zdeg = jnp.zeros((_ZROWS, 16), jnp.float32)
    zrows = jnp.zeros((_ZROWS, _D), jnp.float32)

    degs = _deg_kernel(src_r, dst_r, zdeg)
    h1, n_in, n_out = _norm_call(degs, x)
    p1 = _agg_kernel(h1, src_r, dst_r, zrows)
    h2 = _layer_call(p1, n_in, n_out, W1, b1.reshape(1, _D), relu=True)
    p2 = _agg_kernel(h2, src_r, dst_r, zrows)
    return _layer_call(p2, n_in, n_out, W2, b2.reshape(1, _D), relu=False)
